# baseline (speedup 1.0000x reference)
import functools

import jax
import jax.numpy as jnp
from jax.experimental import pallas as pl
from jax.experimental.pallas import tpu as pltpu

_T = 512
_D = 256


def _xorsig_kernel(x_ref, base_ref, dpad_ref, dist_ref, idx_ref,
                   p1_ref, p2_ref, stc_ref):
    bn = x_ref.shape[0]

    @pl.when(pl.program_id(0) == 0)
    def _build_sigs():
        b = jnp.sign(base_ref[...])
        d = jnp.sign(dpad_ref[...])
        sig = jnp.where(d == 0.0, b, d)
        one = jnp.float32(1.0)
        p1_ref[...] = (one - 2.0 * (sig > 0.0)).astype(jnp.int8)
        p2_ref[...] = (one - 2.0 * (sig < 0.0)).astype(jnp.int8)
        st = jax.lax.dot_general(
            jnp.ones((1, _D), dtype=jnp.bfloat16),
            jnp.abs(sig).astype(jnp.bfloat16),
            dimension_numbers=(((1,), (1,)), ((), ())),
            preferred_element_type=jnp.float32)
        iota = jax.lax.broadcasted_iota(jnp.int32, (1, _T), 1)
        stc_ref[...] = st.astype(jnp.int32) * 512 + iota

    x = x_ref[...]
    posb = (x > 0.0).astype(jnp.int8)
    negb = (x < 0.0).astype(jnp.int8)
    t1 = jax.lax.dot_general(
        posb, p1_ref[...],
        dimension_numbers=(((1,), (1,)), ((), ())),
        preferred_element_type=jnp.int32)
    t2 = jax.lax.dot_general(
        negb, p2_ref[...],
        dimension_numbers=(((1,), (1,)), ((), ())),
        preferred_element_type=jnp.int32)
    comb = stc_ref[...] + ((t1 + t2) << 9)
    dist_ref[...] = comb >> 9
    minv = jnp.min(comb, axis=1, keepdims=True)
    idx_ref[...] = minv & 511


@functools.partial(jax.jit, static_argnames=("block_n",))
def _xorsig(x2, base2, dpad, block_n):
    n = x2.shape[0]
    grid = (n // block_n,)
    dist, idx = pl.pallas_call(
        _xorsig_kernel,
        grid=grid,
        in_specs=[
            pl.BlockSpec((block_n, _D), lambda i: (i, 0)),
            pl.BlockSpec((1, _D), lambda i: (0, 0)),
            pl.BlockSpec((_T, _D), lambda i: (0, 0)),
        ],
        out_specs=[
            pl.BlockSpec((block_n, _T), lambda i: (i, 0)),
            pl.BlockSpec((block_n, 1), lambda i: (i, 0)),
        ],
        out_shape=[
            jax.ShapeDtypeStruct((n, _T), jnp.int32),
            jax.ShapeDtypeStruct((n, 1), jnp.int32),
        ],
        scratch_shapes=[
            pltpu.VMEM((_T, _D), jnp.int8),
            pltpu.VMEM((_T, _D), jnp.int8),
            pltpu.VMEM((1, _T), jnp.int32),
        ],
    )(x2, base2, dpad)
    return dist, idx


def kernel(x, base, deltas):
    batch_shape = x.shape[:-1]
    dim = base.shape[0]
    x2 = x.reshape(-1, dim)
    base2 = base.reshape(1, dim)
    dpad = jnp.concatenate(
        [jnp.zeros((1, dim), deltas.dtype), deltas], axis=0)
    dist, idx = _xorsig(x2, base2, dpad, 4096)
    distances = dist.reshape(*batch_shape, _T)
    tile_idx = idx.reshape(*batch_shape)
    return (tile_idx, distances)


# speedup vs baseline: 1.2246x; 1.2246x over previous
import functools

import jax
import jax.numpy as jnp
from jax.experimental import pallas as pl
from jax.experimental.pallas import tpu as pltpu

_T = 512
_D = 256


def _xorsig_kernel(x_ref, base_ref, dpad_ref, dist_ref, idx_ref,
                   p_ref, stc_ref):
    bn = x_ref.shape[0]

    @pl.when(pl.program_id(0) == 0)
    def _build_sigs():
        b = jnp.sign(base_ref[...])
        d = jnp.sign(dpad_ref[...])
        sig = jnp.where(d == 0.0, b, d)
        one = jnp.float32(1.0)
        p_ref[...] = jnp.concatenate(
            [(one - 2.0 * (sig > 0.0)).astype(jnp.bfloat16),
             (one - 2.0 * (sig < 0.0)).astype(jnp.bfloat16)], axis=1)
        st = jax.lax.dot_general(
            jnp.ones((1, _D), dtype=jnp.bfloat16),
            jnp.abs(sig).astype(jnp.bfloat16),
            dimension_numbers=(((1,), (1,)), ((), ())),
            preferred_element_type=jnp.float32)
        iota = jax.lax.broadcasted_iota(jnp.int32, (1, _T), 1).astype(jnp.float32)
        stc_ref[...] = st + iota * (1.0 / 512.0)

    x = x_ref[...]
    a = jnp.concatenate([(x > 0.0).astype(jnp.bfloat16),
                         (x < 0.0).astype(jnp.bfloat16)], axis=1)
    t = jax.lax.dot_general(
        a, p_ref[...],
        dimension_numbers=(((1,), (1,)), ((), ())),
        preferred_element_type=jnp.float32)
    comb = stc_ref[...] + t
    dist_ref[...] = comb.astype(jnp.int32)
    minv = jnp.min(comb, axis=1, keepdims=True)
    mind = minv.astype(jnp.int32)
    idx_ref[...] = ((minv - mind.astype(jnp.float32)) * 512.0
                    ).astype(jnp.int32)


@functools.partial(jax.jit, static_argnames=("block_n",))
def _xorsig(x2, base2, dpad, block_n):
    n = x2.shape[0]
    grid = (n // block_n,)
    dist, idx = pl.pallas_call(
        _xorsig_kernel,
        grid=grid,
        in_specs=[
            pl.BlockSpec((block_n, _D), lambda i: (i, 0)),
            pl.BlockSpec((1, _D), lambda i: (0, 0)),
            pl.BlockSpec((_T, _D), lambda i: (0, 0)),
        ],
        out_specs=[
            pl.BlockSpec((block_n, _T), lambda i: (i, 0)),
            pl.BlockSpec((block_n, 1), lambda i: (i, 0)),
        ],
        out_shape=[
            jax.ShapeDtypeStruct((n, _T), jnp.int32),
            jax.ShapeDtypeStruct((n, 1), jnp.int32),
        ],
        scratch_shapes=[
            pltpu.VMEM((_T, 2 * _D), jnp.bfloat16),
            pltpu.VMEM((1, _T), jnp.float32),
        ],
    )(x2, base2, dpad)
    return dist, idx


def kernel(x, base, deltas):
    batch_shape = x.shape[:-1]
    dim = base.shape[0]
    x2 = x.reshape(-1, dim)
    base2 = base.reshape(1, dim)
    dpad = jnp.concatenate(
        [jnp.zeros((1, dim), deltas.dtype), deltas], axis=0)
    dist, idx = _xorsig(x2, base2, dpad, 4096)
    distances = dist.reshape(*batch_shape, _T)
    tile_idx = idx.reshape(*batch_shape)
    return (tile_idx, distances)


# speedup vs baseline: 1.3557x; 1.1070x over previous
import functools

import jax
import jax.numpy as jnp
from jax.experimental import pallas as pl
from jax.experimental.pallas import tpu as pltpu

_T = 512
_D = 256


def _xorsig_kernel(x_ref, base_ref, dlt_ref, dist_ref, idx_ref,
                   p_ref, stc_ref):
    bn = x_ref.shape[0]

    @pl.when(pl.program_id(0) == 0)
    def _build_sigs():
        b = jnp.sign(base_ref[...])
        d = jnp.sign(dlt_ref[...])
        sig = jnp.concatenate(
            [b, jnp.where(d == 0.0, b, d)], axis=0)
        one = jnp.float32(1.0)
        p_ref[...] = jnp.concatenate(
            [(one - 2.0 * (sig > 0.0)).astype(jnp.bfloat16),
             (one - 2.0 * (sig < 0.0)).astype(jnp.bfloat16)], axis=1)
        st = jax.lax.dot_general(
            jnp.ones((1, _D), dtype=jnp.bfloat16),
            jnp.abs(sig).astype(jnp.bfloat16),
            dimension_numbers=(((1,), (1,)), ((), ())),
            preferred_element_type=jnp.float32)
        iota = jax.lax.broadcasted_iota(jnp.int32, (1, _T), 1).astype(jnp.float32)
        stc_ref[...] = st + iota * (1.0 / 512.0)

    x = x_ref[...]
    a = jnp.concatenate([(x > 0.0).astype(jnp.bfloat16),
                         (x < 0.0).astype(jnp.bfloat16)], axis=1)
    t = jax.lax.dot_general(
        a, p_ref[...],
        dimension_numbers=(((1,), (1,)), ((), ())),
        preferred_element_type=jnp.float32)
    comb = stc_ref[...] + t
    dist_ref[...] = comb.astype(jnp.int32)
    minv = jnp.min(comb, axis=1, keepdims=True)
    mind = minv.astype(jnp.int32)
    idx_ref[...] = ((minv - mind.astype(jnp.float32)) * 512.0
                    ).astype(jnp.int32)


@functools.partial(jax.jit, static_argnames=("block_n",))
def _xorsig(x2, base2, deltas, block_n):
    n = x2.shape[0]
    grid = (n // block_n,)
    dist, idx = pl.pallas_call(
        _xorsig_kernel,
        grid=grid,
        in_specs=[
            pl.BlockSpec((block_n, _D), lambda i: (i, 0)),
            pl.BlockSpec((1, _D), lambda i: (0, 0)),
            pl.BlockSpec((_T - 1, _D), lambda i: (0, 0)),
        ],
        out_specs=[
            pl.BlockSpec((block_n, _T), lambda i: (i, 0)),
            pl.BlockSpec((block_n, 1), lambda i: (i, 0)),
        ],
        out_shape=[
            jax.ShapeDtypeStruct((n, _T), jnp.int32),
            jax.ShapeDtypeStruct((n, 1), jnp.int32),
        ],
        scratch_shapes=[
            pltpu.VMEM((_T, 2 * _D), jnp.bfloat16),
            pltpu.VMEM((1, _T), jnp.float32),
        ],
    )(x2, base2, deltas)
    return dist, idx


def kernel(x, base, deltas):
    batch_shape = x.shape[:-1]
    dim = base.shape[0]
    x2 = x.reshape(-1, dim)
    base2 = base.reshape(1, dim)
    dist, idx = _xorsig(x2, base2, deltas, 4096)
    distances = dist.reshape(*batch_shape, _T)
    tile_idx = idx.reshape(*batch_shape)
    return (tile_idx, distances)


# speedup vs baseline: 1.7838x; 1.3157x over previous
import functools

import jax
import jax.numpy as jnp
from jax.experimental import pallas as pl
from jax.experimental.pallas import tpu as pltpu

_T = 512
_D = 256


def _xorsig_kernel(x_ref, base_ref, dlt_ref, dist_ref, idx_ref,
                   p_ref, stc_ref):
    bn = x_ref.shape[0]

    @pl.when(pl.program_id(0) == 0)
    def _build_sigs():
        b = jnp.sign(base_ref[...])
        d = jnp.sign(dlt_ref[...])
        sig = jnp.concatenate(
            [b, jnp.where(d == 0.0, b, d)], axis=0)
        one = jnp.float32(1.0)
        p_ref[...] = jnp.concatenate(
            [(one - 2.0 * (sig > 0.0)).astype(jnp.bfloat16),
             (one - 2.0 * (sig < 0.0)).astype(jnp.bfloat16)], axis=1)
        st = jax.lax.dot_general(
            jnp.ones((1, _D), dtype=jnp.bfloat16),
            jnp.abs(sig).astype(jnp.bfloat16),
            dimension_numbers=(((1,), (1,)), ((), ())),
            preferred_element_type=jnp.float32)
        iota = jax.lax.broadcasted_iota(jnp.int32, (1, _T), 1).astype(jnp.float32)
        stc_ref[...] = st + iota * (1.0 / 512.0)

    x = x_ref[...]
    a = jnp.concatenate([(x > 0.0).astype(jnp.bfloat16),
                         (x < 0.0).astype(jnp.bfloat16)], axis=1)
    t = jax.lax.dot_general(
        a, p_ref[...],
        dimension_numbers=(((1,), (1,)), ((), ())),
        preferred_element_type=jnp.float32)
    comb = stc_ref[...] + t
    dist_ref[...] = comb.astype(jnp.int32)
    minv = jnp.min(comb, axis=1, keepdims=True)
    mind = minv.astype(jnp.int32)
    idx = ((minv - mind.astype(jnp.float32)) * 512.0).astype(jnp.int32)
    idx_ref[...] = idx.reshape(bn // 256, 256)


@functools.partial(jax.jit, static_argnames=("block_n",))
def _xorsig(x2, base2, deltas, block_n):
    n = x2.shape[0]
    grid = (n // block_n,)
    dist, idx = pl.pallas_call(
        _xorsig_kernel,
        grid=grid,
        in_specs=[
            pl.BlockSpec((block_n, _D), lambda i: (i, 0)),
            pl.BlockSpec((1, _D), lambda i: (0, 0)),
            pl.BlockSpec((_T - 1, _D), lambda i: (0, 0)),
        ],
        out_specs=[
            pl.BlockSpec((block_n, _T), lambda i: (i, 0)),
            pl.BlockSpec((block_n // 256, 256), lambda i: (i, 0)),
        ],
        out_shape=[
            jax.ShapeDtypeStruct((n, _T), jnp.int32),
            jax.ShapeDtypeStruct((n // 256, 256), jnp.int32),
        ],
        scratch_shapes=[
            pltpu.VMEM((_T, 2 * _D), jnp.bfloat16),
            pltpu.VMEM((1, _T), jnp.float32),
        ],
    )(x2, base2, deltas)
    return dist, idx


def kernel(x, base, deltas):
    batch_shape = x.shape[:-1]
    dim = base.shape[0]
    x2 = x.reshape(-1, dim)
    base2 = base.reshape(1, dim)
    dist, idx = _xorsig(x2, base2, deltas, 4096)
    distances = dist.reshape(*batch_shape, _T)
    tile_idx = idx.reshape(batch_shape)
    return (tile_idx, distances)


# speedup vs baseline: 1.7916x; 1.0044x over previous
import functools

import jax
import jax.numpy as jnp
from jax.experimental import pallas as pl
from jax.experimental.pallas import tpu as pltpu

_T = 512
_D = 256


def _xorsig_kernel(x_ref, base_ref, dlt_ref, dist_ref, idx_ref,
                   p_ref, stc_ref):
    bn = x_ref.shape[0]

    @pl.when(pl.program_id(0) == 0)
    def _build_sigs():
        b = jnp.sign(base_ref[...])
        d = jnp.sign(dlt_ref[...])
        sig = jnp.concatenate(
            [b, jnp.where(d == 0.0, b, d)], axis=0)
        one = jnp.float32(1.0)
        p_ref[...] = jnp.concatenate(
            [(one - 2.0 * (sig > 0.0)).astype(jnp.bfloat16),
             (one - 2.0 * (sig < 0.0)).astype(jnp.bfloat16)], axis=1)
        st = jax.lax.dot_general(
            jnp.ones((1, _D), dtype=jnp.bfloat16),
            jnp.abs(sig).astype(jnp.bfloat16),
            dimension_numbers=(((1,), (1,)), ((), ())),
            preferred_element_type=jnp.float32)
        iota = jax.lax.broadcasted_iota(jnp.int32, (1, _T), 1).astype(jnp.float32)
        stc_ref[...] = st + iota * (1.0 / 512.0)

    x = x_ref[...]
    a = jnp.concatenate([(x > 0.0).astype(jnp.bfloat16),
                         (x < 0.0).astype(jnp.bfloat16)], axis=1)
    t = jax.lax.dot_general(
        a, p_ref[...],
        dimension_numbers=(((1,), (1,)), ((), ())),
        preferred_element_type=jnp.float32)
    comb = stc_ref[...] + t
    dist_ref[...] = comb.astype(jnp.int32)
    minv = jnp.min(comb, axis=1, keepdims=True)
    mind = minv.astype(jnp.int32)
    idx = ((minv - mind.astype(jnp.float32)) * 512.0).astype(jnp.int32)
    idx_ref[...] = idx.reshape(bn // 256, 256)


@functools.partial(jax.jit, static_argnames=("block_n",))
def _xorsig(x2, base2, deltas, block_n):
    n = x2.shape[0]
    grid = (n // block_n,)
    dist, idx = pl.pallas_call(
        _xorsig_kernel,
        grid=grid,
        in_specs=[
            pl.BlockSpec((block_n, _D), lambda i: (i, 0)),
            pl.BlockSpec((1, _D), lambda i: (0, 0)),
            pl.BlockSpec((_T - 1, _D), lambda i: (0, 0)),
        ],
        out_specs=[
            pl.BlockSpec((block_n, _T), lambda i: (i, 0)),
            pl.BlockSpec((block_n // 256, 256), lambda i: (i, 0)),
        ],
        out_shape=[
            jax.ShapeDtypeStruct((n, _T), jnp.int32),
            jax.ShapeDtypeStruct((n // 256, 256), jnp.int32),
        ],
        scratch_shapes=[
            pltpu.VMEM((_T, 2 * _D), jnp.bfloat16),
            pltpu.VMEM((1, _T), jnp.float32),
        ],
    )(x2, base2, deltas)
    return dist, idx


def kernel(x, base, deltas):
    batch_shape = x.shape[:-1]
    dim = base.shape[0]
    x2 = x.reshape(-1, dim)
    base2 = base.reshape(1, dim)
    dist, idx = _xorsig(x2, base2, deltas, 2048)
    distances = dist.reshape(*batch_shape, _T)
    tile_idx = idx.reshape(batch_shape)
    return (tile_idx, distances)


# speedup vs baseline: 1.7946x; 1.0017x over previous
import functools

import jax
import jax.numpy as jnp
from jax.experimental import pallas as pl
from jax.experimental.pallas import tpu as pltpu

_T = 512
_D = 256


def _xorsig_kernel(x_ref, base_ref, dlt_ref, dist_ref, idx_ref,
                   p_ref, stc_ref):
    bn = x_ref.shape[0]

    @pl.when(pl.program_id(0) == 0)
    def _build_sigs():
        b = jnp.sign(base_ref[...])
        d = jnp.sign(dlt_ref[...])
        sig = jnp.concatenate(
            [b, jnp.where(d == 0.0, b, d)], axis=0)
        one = jnp.float32(1.0)
        p_ref[...] = jnp.concatenate(
            [(one - 2.0 * (sig > 0.0)).astype(jnp.bfloat16),
             (one - 2.0 * (sig < 0.0)).astype(jnp.bfloat16)], axis=1)
        st = jax.lax.dot_general(
            jnp.ones((1, _D), dtype=jnp.bfloat16),
            jnp.abs(sig).astype(jnp.bfloat16),
            dimension_numbers=(((1,), (1,)), ((), ())),
            preferred_element_type=jnp.float32)
        iota = jax.lax.broadcasted_iota(jnp.int32, (1, _T), 1).astype(jnp.float32)
        stc_ref[...] = st + iota * (1.0 / 512.0)

    x = x_ref[...]
    a = jnp.concatenate([(x > 0.0).astype(jnp.bfloat16),
                         (x < 0.0).astype(jnp.bfloat16)], axis=1)
    mins = []
    for h in range(2):
        th = jax.lax.dot_general(
            a, p_ref[h * (_T // 2):(h + 1) * (_T // 2), :],
            dimension_numbers=(((1,), (1,)), ((), ())),
            preferred_element_type=jnp.float32)
        comb_h = stc_ref[:, h * (_T // 2):(h + 1) * (_T // 2)] + th
        dist_ref[:, h * (_T // 2):(h + 1) * (_T // 2)] = comb_h.astype(jnp.int32)
        mins.append(jnp.min(comb_h, axis=1, keepdims=True))
    minv = jnp.minimum(mins[0], mins[1])
    mind = minv.astype(jnp.int32)
    idx = ((minv - mind.astype(jnp.float32)) * 512.0).astype(jnp.int32)
    idx_ref[...] = idx.reshape(bn // 256, 256)


@functools.partial(jax.jit, static_argnames=("block_n",))
def _xorsig(x2, base2, deltas, block_n):
    n = x2.shape[0]
    grid = (n // block_n,)
    dist, idx = pl.pallas_call(
        _xorsig_kernel,
        grid=grid,
        in_specs=[
            pl.BlockSpec((block_n, _D), lambda i: (i, 0)),
            pl.BlockSpec((1, _D), lambda i: (0, 0)),
            pl.BlockSpec((_T - 1, _D), lambda i: (0, 0)),
        ],
        out_specs=[
            pl.BlockSpec((block_n, _T), lambda i: (i, 0)),
            pl.BlockSpec((block_n // 256, 256), lambda i: (i, 0)),
        ],
        out_shape=[
            jax.ShapeDtypeStruct((n, _T), jnp.int32),
            jax.ShapeDtypeStruct((n // 256, 256), jnp.int32),
        ],
        scratch_shapes=[
            pltpu.VMEM((_T, 2 * _D), jnp.bfloat16),
            pltpu.VMEM((1, _T), jnp.float32),
        ],
    )(x2, base2, deltas)
    return dist, idx


def kernel(x, base, deltas):
    batch_shape = x.shape[:-1]
    dim = base.shape[0]
    x2 = x.reshape(-1, dim)
    base2 = base.reshape(1, dim)
    dist, idx = _xorsig(x2, base2, deltas, 2048)
    distances = dist.reshape(*batch_shape, _T)
    tile_idx = idx.reshape(batch_shape)
    return (tile_idx, distances)
